# baseline (speedup 1.0000x reference)
import functools

import jax
import jax.numpy as jnp
from jax import lax
from jax.experimental import pallas as pl
from jax.experimental.pallas import tpu as pltpu
from jax.experimental.pallas import tpu_sc as plsc

N_NODES = 10000
N_PAD = 10240
F_IN = 128
N_GRAPHS = 64
NC, NS, LANES = 2, 16, 16
NW = NC * NS
E_REAL = 640000 + N_NODES
E_PAD = NW * 128 * (4 * (-(-E_REAL // (NW * 128 * 4))))
N_SC = 10016
DUMMY = 10008
ROWS_PER_TILE = N_SC // NS


def _build_dup_proj(a, C):
    H = a.shape[0]
    S = jnp.zeros((H * C, 16), jnp.float32)
    r = jnp.arange(H * C)
    S = S.at[r, r // C].set(a.reshape(-1))
    S = S.at[r, r // C + 8].set(a.reshape(-1))
    return S


def _build_den_sel(D):
    C = D // 8
    E = jnp.zeros((D + 16, D), jnp.float32)
    r = jnp.arange(D)
    E = E.at[D + r // C, r].set(1.0)
    return E


def _tc_pre_body(x_ref, w_ref, ss_ref, sd_ref, h_ref, ad_ref):
    h = jnp.dot(x_ref[...], w_ref[...], preferred_element_type=jnp.float32)
    h_ref[:, :-16] = h
    h_ref[:, -16:] = jnp.dot(h, ss_ref[...], preferred_element_type=jnp.float32, precision=lax.Precision.HIGHEST)
    ad_ref[...] = jnp.dot(h, sd_ref[...], preferred_element_type=jnp.float32, precision=lax.Precision.HIGHEST)


def _tc_pre(x_pad, W1, S1s, S1d):
    D = W1.shape[1]
    grid = N_PAD // 256
    return pl.pallas_call(
        _tc_pre_body,
        grid=(grid,),
        in_specs=[
            pl.BlockSpec((256, F_IN), lambda i: (i, 0)),
            pl.BlockSpec((F_IN, D), lambda i: (0, 0)),
            pl.BlockSpec((D, 16), lambda i: (0, 0)),
            pl.BlockSpec((D, 16), lambda i: (0, 0)),
        ],
        out_specs=[
            pl.BlockSpec((256, D + 16), lambda i: (i, 0)),
            pl.BlockSpec((256, 16), lambda i: (i, 0)),
        ],
        out_shape=[
            jax.ShapeDtypeStruct((N_PAD, D + 16), jnp.float32),
            jax.ShapeDtypeStruct((N_PAD, 16), jnp.float32),
        ],
    )(x_pad, W1, S1s, S1d)


def _tc_mid_body(p0_ref, p1_ref, e1_ref, b1_ref, w2_ref, ss_ref, sd_ref,
                 h_ref, ad_ref):
    acc = p0_ref[...] + p1_ref[...]
    den = jnp.dot(acc, e1_ref[...], preferred_element_type=jnp.float32, precision=lax.Precision.HIGHEST)
    out1 = acc[:, :64] / (den + 1e-16) + b1_ref[...]
    x2 = jnp.where(out1 > 0, out1, jnp.exp(jnp.minimum(out1, 0.0)) - 1.0)
    h2 = jnp.dot(x2, w2_ref[...], preferred_element_type=jnp.float32)
    h_ref[:, :-16] = h2
    h_ref[:, -16:] = jnp.dot(h2, ss_ref[...], preferred_element_type=jnp.float32, precision=lax.Precision.HIGHEST)
    ad_ref[...] = jnp.dot(h2, sd_ref[...], preferred_element_type=jnp.float32, precision=lax.Precision.HIGHEST)


def _tc_mid(p0, p1, E1, b1row, W2, S2s, S2d):
    grid = N_PAD // 256
    return pl.pallas_call(
        _tc_mid_body,
        grid=(grid,),
        in_specs=[
            pl.BlockSpec((256, 80), lambda i: (i, 0)),
            pl.BlockSpec((256, 80), lambda i: (i, 0)),
            pl.BlockSpec((80, 64), lambda i: (0, 0)),
            pl.BlockSpec((1, 64), lambda i: (0, 0)),
            pl.BlockSpec((64, 128), lambda i: (0, 0)),
            pl.BlockSpec((128, 16), lambda i: (0, 0)),
            pl.BlockSpec((128, 16), lambda i: (0, 0)),
        ],
        out_specs=[
            pl.BlockSpec((256, 144), lambda i: (i, 0)),
            pl.BlockSpec((256, 16), lambda i: (i, 0)),
        ],
        out_shape=[
            jax.ShapeDtypeStruct((N_PAD, 144), jnp.float32),
            jax.ShapeDtypeStruct((N_PAD, 16), jnp.float32),
        ],
    )(p0, p1, E1, b1row, W2, S2s, S2d)


def _tc_fin_body(p0_ref, p1_ref, e2_ref, mm_ref, b2_ref, batch_ref,
                 fcw_ref, fcb_ref, out_ref, sum_ref, cnt_ref):
    i = pl.program_id(0)
    acc = p0_ref[...] + p1_ref[...]
    den = jnp.dot(acc, e2_ref[...], preferred_element_type=jnp.float32, precision=lax.Precision.HIGHEST)
    h2n = acc[:, :128] / (den + 1e-16)
    val = jnp.dot(h2n, mm_ref[...], preferred_element_type=jnp.float32, precision=lax.Precision.HIGHEST) + b2_ref[...]
    bt = batch_ref[...].reshape(1, 256)
    ohT = (lax.broadcasted_iota(jnp.int32, (N_GRAPHS, 256), 0) == bt
           ).astype(jnp.float32)
    pooled = jnp.dot(ohT, val, preferred_element_type=jnp.float32, precision=lax.Precision.HIGHEST)
    cnt = jnp.dot(ohT, jnp.ones((256, 16), jnp.float32),
                  preferred_element_type=jnp.float32,
                  precision=lax.Precision.HIGHEST)
    prev_s = jnp.where(i == 0, 0.0, sum_ref[...])
    prev_c = jnp.where(i == 0, 0.0, cnt_ref[...])
    sum_ref[...] = prev_s + pooled
    cnt_ref[...] = prev_c + cnt

    @pl.when(i == pl.num_programs(0) - 1)
    def _():
        tot = sum_ref[...] / jnp.maximum(cnt_ref[...], 1.0)
        out_ref[...] = jnp.dot(tot, fcw_ref[...],
                               preferred_element_type=jnp.float32,
                               precision=lax.Precision.HIGHEST) + fcb_ref[...]


def _tc_fin(p0, p1, E2, Mmean, b2row, batch3, fcw_pad, fcb_pad):
    grid = N_PAD // 256
    return pl.pallas_call(
        _tc_fin_body,
        grid=(grid,),
        in_specs=[
            pl.BlockSpec((256, 144), lambda i: (i, 0)),
            pl.BlockSpec((256, 144), lambda i: (i, 0)),
            pl.BlockSpec((144, 128), lambda i: (0, 0)),
            pl.BlockSpec((128, 16), lambda i: (0, 0)),
            pl.BlockSpec((1, 16), lambda i: (0, 0)),
            pl.BlockSpec((1, 1, 256), lambda i: (i, 0, 0)),
            pl.BlockSpec((16, 128), lambda i: (0, 0)),
            pl.BlockSpec((1, 128), lambda i: (0, 0)),
        ],
        out_specs=pl.BlockSpec((N_GRAPHS, 128), lambda i: (0, 0)),
        out_shape=jax.ShapeDtypeStruct((N_GRAPHS, 128), jnp.float32),
        scratch_shapes=[
            pltpu.VMEM((N_GRAPHS, 16), jnp.float32),
            pltpu.VMEM((N_GRAPHS, 16), jnp.float32),
        ],
    )(p0, p1, E2, Mmean, b2row, batch3, fcw_pad, fcb_pad)


def _make_sc_edge(D, K):
    W_ACC = D + 16
    n_chunks = D // 16
    CPW = E_PAD // (NW * K)
    row_chunks = [(t * K, K) for t in range(ROWS_PER_TILE // K)]
    if ROWS_PER_TILE % K:
        row_chunks.append((ROWS_PER_TILE - ROWS_PER_TILE % K,
                           ROWS_PER_TILE % K))

    def body(src_hbm, dst_hbm, htab, adtab, out_hbm,
             s0, s1, s2, s3, d0, d1, d2, d3,
             hr0, hr1, ad0, ad1, m0, m1, acc,
             semh0, semh1, sema0, sema1, sems0, sems1):
        cid = lax.axis_index("c")
        sid = lax.axis_index("s")
        wid = sid * NC + cid
        sidx = (s0, s1, s2, s3)
        didx = (d0, d1, d2, d3)
        hrows = (hr0, hr1)
        aad = (ad0, ad1)
        msg = (m0, m1)
        semH = (semh0, semh1)
        semA = (sema0, sema1)
        semS = (sems0, sems1)

        def zrow(r, carry):
            for j in range(W_ACC // 16):
                m0[r, pl.ds(j * 16, 16)] = jnp.zeros((16,), jnp.float32)
            return carry

        lax.fori_loop(0, K, zrow, 0)
        for off, n in row_chunks:
            pltpu.sync_copy(m0.at[pl.ds(0, n)],
                            acc.at[pl.ds(sid * ROWS_PER_TILE + off, n)])
        plsc.subcore_barrier()

        iota = lax.iota(jnp.int32, LANES)

        def load_idx(g, s):
            row = wid * CPW + g
            pltpu.sync_copy(src_hbm.at[row], sidx[s])
            pltpu.sync_copy(dst_hbm.at[row], didx[s])

        def start_g(s, b):
            pltpu.async_copy(htab.at[sidx[s]], hrows[b], semH[b])
            pltpu.async_copy(adtab.at[didx[s]], aad[b], semA[b])

        def wait_g(s, b):
            pltpu.make_async_copy(htab.at[sidx[s]], hrows[b], semH[b]).wait()
            pltpu.make_async_copy(adtab.at[didx[s]], aad[b], semA[b]).wait()

        def make_edge(b):
            hb, ab, mb = hrows[b], aad[b], msg[b]

            def edge(e):
                z = hb[e, pl.ds(D, 16)] + ab[e]
                z = jnp.where(z >= 0.0, z, z * jnp.float32(0.2))
                w = jnp.exp(z)
                mb[e, pl.ds(D, 16)] = w
                for j in range(n_chunks):
                    h16 = hb[e, pl.ds(j * 16, 16)]
                    if D == 64:
                        wexp = jnp.where(iota < 8, w[2 * j], w[2 * j + 1])
                        mb[e, pl.ds(j * 16, 16)] = h16 * wexp
                    else:
                        mb[e, pl.ds(j * 16, 16)] = h16 * w[j]

            return edge

        edges = (make_edge(0), make_edge(1))

        load_idx(0, 0)
        start_g(0, 0)

        def outer(go, carry):
            for b in range(4):
                g = go * 4 + b
                db = b % 2
                ns = (b + 1) % 4
                nb = (b + 1) % 2

                @pl.when(g + 1 < CPW)
                def _():
                    load_idx(g + 1, ns)
                    start_g(ns, nb)

                @pl.when(g >= 2)
                def _():
                    pltpu.make_async_copy(
                        msg[db], acc.at[didx[(b + 2) % 4]], semS[db]).wait()

                wait_g(b, db)
                plsc.parallel_loop(0, K, 1, unroll=4)(edges[db])
                pltpu.async_copy(msg[db], acc.at[didx[b]], semS[db], add=True)
            return carry

        lax.fori_loop(0, CPW // 4, outer, 0)
        pltpu.make_async_copy(msg[0], acc.at[didx[2]], semS[0]).wait()
        pltpu.make_async_copy(msg[1], acc.at[didx[3]], semS[1]).wait()
        plsc.subcore_barrier()
        for off, n in row_chunks:
            r = sid * ROWS_PER_TILE + off
            pltpu.sync_copy(acc.at[pl.ds(r, n)], out_hbm.at[cid].at[pl.ds(r, n)])

    mesh = plsc.VectorSubcoreMesh(core_axis_name="c", subcore_axis_name="s",
                                  num_cores=NC, num_subcores=NS)
    return pl.kernel(
        body,
        out_type=jax.ShapeDtypeStruct((NC, N_SC, W_ACC), jnp.float32),
        mesh=mesh,
        compiler_params=pltpu.CompilerParams(use_tc_tiling_on_sc=False),
        scratch_types=(
            [pltpu.VMEM((K,), jnp.int32)] * 8
            + [
                pltpu.VMEM((K, W_ACC), jnp.float32),
                pltpu.VMEM((K, W_ACC), jnp.float32),
                pltpu.VMEM((K, 16), jnp.float32),
                pltpu.VMEM((K, 16), jnp.float32),
                pltpu.VMEM((K, W_ACC), jnp.float32),
                pltpu.VMEM((K, W_ACC), jnp.float32),
                pltpu.VMEM_SHARED((N_SC, W_ACC), jnp.float32),
            ]
            + [pltpu.SemaphoreType.DMA] * 6
        ),
    )


@functools.lru_cache(maxsize=None)
def _sc_edge(D, K):
    return _make_sc_edge(D, K)


def kernel(x, edge_index, batch, W1, a1_src, a1_dst, b1,
           W2, a2_src, a2_dst, b2, fc_w, fc_b):
    f32 = jnp.float32
    x_pad = jnp.zeros((N_PAD, F_IN), f32).at[:N_NODES].set(x)

    loop_idx = jnp.arange(N_NODES, dtype=jnp.int32)
    src = jnp.concatenate([edge_index[0], loop_idx])
    dst = jnp.concatenate([edge_index[1], loop_idx])
    src1d = jnp.full((E_PAD,), DUMMY, jnp.int32).at[:E_REAL].set(src)
    dst1d = jnp.full((E_PAD,), DUMMY, jnp.int32).at[:E_REAL].set(dst)

    S1s = _build_dup_proj(a1_src, 8)
    S1d = _build_dup_proj(a1_dst, 8)
    S2s = _build_dup_proj(a2_src, 16)
    S2d = _build_dup_proj(a2_dst, 16)
    E1 = _build_den_sel(64)
    E2 = _build_den_sel(128)
    r128 = jnp.arange(128)
    Mmean = jnp.zeros((128, 16), f32).at[r128, r128 % 16].set(1.0 / 8.0)
    b1row = b1.reshape(1, 64)
    b2row = b2.reshape(1, 16)
    fcw_pad = jnp.zeros((16, 128), f32).at[:, :4].set(fc_w)
    fcb_pad = jnp.zeros((1, 128), f32).at[0, :4].set(fc_b)
    batch3 = jnp.concatenate(
        [batch, jnp.full((N_PAD - N_NODES,), -1, jnp.int32)]).reshape(-1, 1, 256)

    h1, ad1 = _tc_pre(x_pad, W1, S1s, S1d)
    part1 = _sc_edge(64, 128)(src1d.reshape(-1, 128), dst1d.reshape(-1, 128),
                              h1, ad1)
    part1 = jnp.pad(part1, ((0, 0), (0, N_PAD - N_SC), (0, 0)))
    h2, ad2 = _tc_mid(part1[0], part1[1], E1, b1row, W2, S2s, S2d)
    part2 = _sc_edge(128, 64)(src1d.reshape(-1, 64), dst1d.reshape(-1, 64),
                              h2, ad2)
    part2 = jnp.pad(part2, ((0, 0), (0, N_PAD - N_SC), (0, 0)))
    out = _tc_fin(part2[0], part2[1], E2, Mmean, b2row, batch3, fcw_pad, fcb_pad)
    return out[:, :4]


# speedup vs baseline: 1.1628x; 1.1628x over previous
import functools

import jax
import jax.numpy as jnp
from jax import lax
from jax.experimental import pallas as pl
from jax.experimental.pallas import tpu as pltpu
from jax.experimental.pallas import tpu_sc as plsc

N_NODES = 10000
N_PAD = 10240
F_IN = 128
N_GRAPHS = 64
NC, NS, LANES = 2, 16, 16
NW = NC * NS
E_REAL = 640000 + N_NODES
E_PAD = NW * 128 * (4 * (-(-E_REAL // (NW * 128 * 4))))
N_SC = 10016
DUMMY = 10008
ROWS_PER_TILE = N_SC // NS


def _build_dup_proj(a, C):
    H = a.shape[0]
    S = jnp.zeros((H * C, 16), jnp.float32)
    r = jnp.arange(H * C)
    S = S.at[r, r // C].set(a.reshape(-1))
    S = S.at[r, r // C + 8].set(a.reshape(-1))
    return S


def _build_den_sel(D):
    C = D // 8
    E = jnp.zeros((D + 16, D), jnp.float32)
    r = jnp.arange(D)
    E = E.at[D + r // C, r].set(1.0)
    return E


def _tc_pre_body(x_ref, w_ref, ss_ref, sd_ref, h_ref, ad_ref):
    h = jnp.dot(x_ref[...], w_ref[...], preferred_element_type=jnp.float32)
    h_ref[:, :-16] = h
    h_ref[:, -16:] = jnp.dot(h, ss_ref[...], preferred_element_type=jnp.float32, precision=lax.Precision.HIGHEST)
    ad_ref[...] = jnp.dot(h, sd_ref[...], preferred_element_type=jnp.float32, precision=lax.Precision.HIGHEST)


def _tc_pre(x_pad, W1, S1s, S1d):
    D = W1.shape[1]
    grid = N_PAD // 256
    return pl.pallas_call(
        _tc_pre_body,
        grid=(grid,),
        in_specs=[
            pl.BlockSpec((256, F_IN), lambda i: (i, 0)),
            pl.BlockSpec((F_IN, D), lambda i: (0, 0)),
            pl.BlockSpec((D, 16), lambda i: (0, 0)),
            pl.BlockSpec((D, 16), lambda i: (0, 0)),
        ],
        out_specs=[
            pl.BlockSpec((256, D + 16), lambda i: (i, 0)),
            pl.BlockSpec((256, 16), lambda i: (i, 0)),
        ],
        out_shape=[
            jax.ShapeDtypeStruct((N_PAD, D + 16), jnp.float32),
            jax.ShapeDtypeStruct((N_PAD, 16), jnp.float32),
        ],
    )(x_pad, W1, S1s, S1d)


def _tc_mid_body(p0_ref, p1_ref, e1_ref, b1_ref, w2_ref, ss_ref, sd_ref,
                 h_ref, ad_ref):
    acc = p0_ref[...] + p1_ref[...]
    den = jnp.dot(acc, e1_ref[...], preferred_element_type=jnp.float32, precision=lax.Precision.HIGHEST)
    out1 = acc[:, :64] / (den + 1e-16) + b1_ref[...]
    x2 = jnp.where(out1 > 0, out1, jnp.exp(jnp.minimum(out1, 0.0)) - 1.0)
    h2 = jnp.dot(x2, w2_ref[...], preferred_element_type=jnp.float32)
    h_ref[:, :-16] = h2
    h_ref[:, -16:] = jnp.dot(h2, ss_ref[...], preferred_element_type=jnp.float32, precision=lax.Precision.HIGHEST)
    ad_ref[...] = jnp.dot(h2, sd_ref[...], preferred_element_type=jnp.float32, precision=lax.Precision.HIGHEST)


def _tc_mid(p0, p1, E1, b1row, W2, S2s, S2d):
    grid = N_PAD // 256
    return pl.pallas_call(
        _tc_mid_body,
        grid=(grid,),
        in_specs=[
            pl.BlockSpec((256, 80), lambda i: (i, 0)),
            pl.BlockSpec((256, 80), lambda i: (i, 0)),
            pl.BlockSpec((80, 64), lambda i: (0, 0)),
            pl.BlockSpec((1, 64), lambda i: (0, 0)),
            pl.BlockSpec((64, 128), lambda i: (0, 0)),
            pl.BlockSpec((128, 16), lambda i: (0, 0)),
            pl.BlockSpec((128, 16), lambda i: (0, 0)),
        ],
        out_specs=[
            pl.BlockSpec((256, 144), lambda i: (i, 0)),
            pl.BlockSpec((256, 16), lambda i: (i, 0)),
        ],
        out_shape=[
            jax.ShapeDtypeStruct((N_PAD, 144), jnp.float32),
            jax.ShapeDtypeStruct((N_PAD, 16), jnp.float32),
        ],
    )(p0, p1, E1, b1row, W2, S2s, S2d)


def _tc_fin_body(p0_ref, p1_ref, e2_ref, mm_ref, b2_ref, batch_ref,
                 fcw_ref, fcb_ref, out_ref, sum_ref, cnt_ref):
    i = pl.program_id(0)
    acc = p0_ref[...] + p1_ref[...]
    den = jnp.dot(acc, e2_ref[...], preferred_element_type=jnp.float32, precision=lax.Precision.HIGHEST)
    h2n = acc[:, :128] / (den + 1e-16)
    val = jnp.dot(h2n, mm_ref[...], preferred_element_type=jnp.float32, precision=lax.Precision.HIGHEST) + b2_ref[...]
    bt = batch_ref[...].reshape(1, 256)
    ohT = (lax.broadcasted_iota(jnp.int32, (N_GRAPHS, 256), 0) == bt
           ).astype(jnp.float32)
    pooled = jnp.dot(ohT, val, preferred_element_type=jnp.float32, precision=lax.Precision.HIGHEST)
    cnt = jnp.dot(ohT, jnp.ones((256, 16), jnp.float32),
                  preferred_element_type=jnp.float32,
                  precision=lax.Precision.HIGHEST)
    prev_s = jnp.where(i == 0, 0.0, sum_ref[...])
    prev_c = jnp.where(i == 0, 0.0, cnt_ref[...])
    sum_ref[...] = prev_s + pooled
    cnt_ref[...] = prev_c + cnt

    @pl.when(i == pl.num_programs(0) - 1)
    def _():
        tot = sum_ref[...] / jnp.maximum(cnt_ref[...], 1.0)
        out_ref[...] = jnp.dot(tot, fcw_ref[...],
                               preferred_element_type=jnp.float32,
                               precision=lax.Precision.HIGHEST) + fcb_ref[...]


def _tc_fin(p0, p1, E2, Mmean, b2row, batch3, fcw_pad, fcb_pad):
    grid = N_PAD // 256
    return pl.pallas_call(
        _tc_fin_body,
        grid=(grid,),
        in_specs=[
            pl.BlockSpec((256, 144), lambda i: (i, 0)),
            pl.BlockSpec((256, 144), lambda i: (i, 0)),
            pl.BlockSpec((144, 128), lambda i: (0, 0)),
            pl.BlockSpec((128, 16), lambda i: (0, 0)),
            pl.BlockSpec((1, 16), lambda i: (0, 0)),
            pl.BlockSpec((1, 1, 256), lambda i: (i, 0, 0)),
            pl.BlockSpec((16, 128), lambda i: (0, 0)),
            pl.BlockSpec((1, 128), lambda i: (0, 0)),
        ],
        out_specs=pl.BlockSpec((N_GRAPHS, 128), lambda i: (0, 0)),
        out_shape=jax.ShapeDtypeStruct((N_GRAPHS, 128), jnp.float32),
        scratch_shapes=[
            pltpu.VMEM((N_GRAPHS, 16), jnp.float32),
            pltpu.VMEM((N_GRAPHS, 16), jnp.float32),
        ],
    )(p0, p1, E2, Mmean, b2row, batch3, fcw_pad, fcb_pad)


def _make_sc_edge(D, K):
    W_ACC = D + 16
    n_chunks = D // 16
    CPW = E_PAD // (NW * K)
    row_chunks = [(t * K, K) for t in range(ROWS_PER_TILE // K)]
    if ROWS_PER_TILE % K:
        row_chunks.append((ROWS_PER_TILE - ROWS_PER_TILE % K,
                           ROWS_PER_TILE % K))

    def body(src_hbm, dst_hbm, htab, adtab, out_hbm,
             s0, s1, s2, s3, d0, d1, d2, d3,
             hr0, hr1, ad0, ad1, m0, m1, acc,
             semh0, semh1, sema0, sema1, sems0, sems1,
             semi0, semi1, semi2, semi3):
        cid = lax.axis_index("c")
        sid = lax.axis_index("s")
        wid = sid * NC + cid
        sidx = (s0, s1, s2, s3)
        didx = (d0, d1, d2, d3)
        hrows = (hr0, hr1)
        aad = (ad0, ad1)
        msg = (m0, m1)
        semH = (semh0, semh1)
        semA = (sema0, sema1)
        semS = (sems0, sems1)
        semI = (semi0, semi1, semi2, semi3)

        def zrow(r, carry):
            for j in range(W_ACC // 16):
                m0[r, pl.ds(j * 16, 16)] = jnp.zeros((16,), jnp.float32)
            return carry

        lax.fori_loop(0, K, zrow, 0)
        for off, n in row_chunks:
            pltpu.sync_copy(m0.at[pl.ds(0, n)],
                            acc.at[pl.ds(sid * ROWS_PER_TILE + off, n)])
        plsc.subcore_barrier()

        iota = lax.iota(jnp.int32, LANES)

        def load_idx(g, s):
            row = wid * CPW + g
            pltpu.async_copy(src_hbm.at[row], sidx[s], semI[s])
            pltpu.async_copy(dst_hbm.at[row], didx[s], semI[s])

        def wait_idx(g, s):
            row = wid * CPW + g
            pltpu.make_async_copy(src_hbm.at[row], sidx[s], semI[s]).wait()
            pltpu.make_async_copy(dst_hbm.at[row], didx[s], semI[s]).wait()

        def start_g(s, b):
            pltpu.async_copy(htab.at[sidx[s]], hrows[b], semH[b])
            pltpu.async_copy(adtab.at[didx[s]], aad[b], semA[b])

        def wait_g(s, b):
            pltpu.make_async_copy(htab.at[sidx[s]], hrows[b], semH[b]).wait()
            pltpu.make_async_copy(adtab.at[didx[s]], aad[b], semA[b]).wait()

        def make_edge(b):
            hb, ab, mb = hrows[b], aad[b], msg[b]

            def edge(e):
                z = hb[e, pl.ds(D, 16)] + ab[e]
                z = jnp.where(z >= 0.0, z, z * jnp.float32(0.2))
                w = jnp.exp(z)
                mb[e, pl.ds(D, 16)] = w
                for j in range(n_chunks):
                    h16 = hb[e, pl.ds(j * 16, 16)]
                    if D == 64:
                        wexp = jnp.where(iota < 8, w[2 * j], w[2 * j + 1])
                        mb[e, pl.ds(j * 16, 16)] = h16 * wexp
                    else:
                        mb[e, pl.ds(j * 16, 16)] = h16 * w[j]

            return edge

        edges = (make_edge(0), make_edge(1))

        load_idx(0, 0)
        wait_idx(0, 0)
        load_idx(1, 1)
        start_g(0, 0)

        def outer(go, carry):
            for b in range(4):
                g = go * 4 + b
                db = b % 2
                ns = (b + 1) % 4
                nb = (b + 1) % 2

                @pl.when(g >= 2)
                def _():
                    pltpu.make_async_copy(
                        msg[db], acc.at[didx[(b + 2) % 4]], semS[db]).wait()

                @pl.when(g + 2 < CPW)
                def _():
                    load_idx(g + 2, (b + 2) % 4)

                @pl.when(g + 1 < CPW)
                def _():
                    wait_idx(g + 1, ns)
                    start_g(ns, nb)

                wait_g(b, db)
                plsc.parallel_loop(0, K, 1, unroll=4)(edges[db])
                pltpu.async_copy(msg[db], acc.at[didx[b]], semS[db], add=True)
            return carry

        lax.fori_loop(0, CPW // 4, outer, 0)
        pltpu.make_async_copy(msg[0], acc.at[didx[2]], semS[0]).wait()
        pltpu.make_async_copy(msg[1], acc.at[didx[3]], semS[1]).wait()
        plsc.subcore_barrier()
        for off, n in row_chunks:
            r = sid * ROWS_PER_TILE + off
            pltpu.sync_copy(acc.at[pl.ds(r, n)], out_hbm.at[cid].at[pl.ds(r, n)])

    mesh = plsc.VectorSubcoreMesh(core_axis_name="c", subcore_axis_name="s",
                                  num_cores=NC, num_subcores=NS)
    return pl.kernel(
        body,
        out_type=jax.ShapeDtypeStruct((NC, N_SC, W_ACC), jnp.float32),
        mesh=mesh,
        compiler_params=pltpu.CompilerParams(use_tc_tiling_on_sc=False),
        scratch_types=(
            [pltpu.VMEM((K,), jnp.int32)] * 8
            + [
                pltpu.VMEM((K, W_ACC), jnp.float32),
                pltpu.VMEM((K, W_ACC), jnp.float32),
                pltpu.VMEM((K, 16), jnp.float32),
                pltpu.VMEM((K, 16), jnp.float32),
                pltpu.VMEM((K, W_ACC), jnp.float32),
                pltpu.VMEM((K, W_ACC), jnp.float32),
                pltpu.VMEM_SHARED((N_SC, W_ACC), jnp.float32),
            ]
            + [pltpu.SemaphoreType.DMA] * 10
        ),
    )


@functools.lru_cache(maxsize=None)
def _sc_edge(D, K):
    return _make_sc_edge(D, K)


def kernel(x, edge_index, batch, W1, a1_src, a1_dst, b1,
           W2, a2_src, a2_dst, b2, fc_w, fc_b):
    f32 = jnp.float32
    x_pad = jnp.zeros((N_PAD, F_IN), f32).at[:N_NODES].set(x)

    loop_idx = jnp.arange(N_NODES, dtype=jnp.int32)
    src = jnp.concatenate([edge_index[0], loop_idx])
    dst = jnp.concatenate([edge_index[1], loop_idx])
    src1d = jnp.full((E_PAD,), DUMMY, jnp.int32).at[:E_REAL].set(src)
    dst1d = jnp.full((E_PAD,), DUMMY, jnp.int32).at[:E_REAL].set(dst)

    S1s = _build_dup_proj(a1_src, 8)
    S1d = _build_dup_proj(a1_dst, 8)
    S2s = _build_dup_proj(a2_src, 16)
    S2d = _build_dup_proj(a2_dst, 16)
    E1 = _build_den_sel(64)
    E2 = _build_den_sel(128)
    r128 = jnp.arange(128)
    Mmean = jnp.zeros((128, 16), f32).at[r128, r128 % 16].set(1.0 / 8.0)
    b1row = b1.reshape(1, 64)
    b2row = b2.reshape(1, 16)
    fcw_pad = jnp.zeros((16, 128), f32).at[:, :4].set(fc_w)
    fcb_pad = jnp.zeros((1, 128), f32).at[0, :4].set(fc_b)
    batch3 = jnp.concatenate(
        [batch, jnp.full((N_PAD - N_NODES,), -1, jnp.int32)]).reshape(-1, 1, 256)

    h1, ad1 = _tc_pre(x_pad, W1, S1s, S1d)
    part1 = _sc_edge(64, 128)(src1d.reshape(-1, 128), dst1d.reshape(-1, 128),
                              h1, ad1)
    part1 = jnp.pad(part1, ((0, 0), (0, N_PAD - N_SC), (0, 0)))
    h2, ad2 = _tc_mid(part1[0], part1[1], E1, b1row, W2, S2s, S2d)
    part2 = _sc_edge(128, 64)(src1d.reshape(-1, 64), dst1d.reshape(-1, 64),
                              h2, ad2)
    part2 = jnp.pad(part2, ((0, 0), (0, N_PAD - N_SC), (0, 0)))
    out = _tc_fin(part2[0], part2[1], E2, Mmean, b2row, batch3, fcw_pad, fcb_pad)
    return out[:, :4]
